# Initial kernel scaffold; baseline (speedup 1.0000x reference)
import jax
import jax.numpy as jnp
from jax.experimental import pallas as pl


def kernel(cls_preds, cls_targets):
    raise NotImplementedError("write your pallas kernel here")


# speedup vs baseline: 42.1839x; 42.1839x over previous
import jax
import jax.numpy as jnp
from jax import lax
from jax.experimental import pallas as pl

NEG2POS_RATIO = 3


def _ohem_body(pt_ref, tgt_ref, out_ref):
    B, N = tgt_ref.shape
    p0 = pt_ref[0]
    p1 = pt_ref[1]
    t = tgt_ref[...]
    pos = t == 1

    d = p1 - p0
    s = jnp.where(pos, d, -d)
    ce = jnp.maximum(-s, 0.0) + jnp.log1p(jnp.exp(-jnp.abs(s)))

    num_pos = jnp.sum(pos.astype(jnp.int32), axis=1, keepdims=True)
    pos_sum = jnp.sum(jnp.where(pos, ce, 0.0))
    cls_loss = jnp.where(pos, 0.0, ce)
    u = lax.bitcast_convert_type(cls_loss, jnp.int32)
    k = jnp.clip(NEG2POS_RATIO * num_pos, 1, N - 1)

    def step(i, T):
        bit = 30 - i
        cand = T | lax.shift_left(jnp.int32(1), bit)
        cnt = jnp.sum((u >= cand).astype(jnp.int32), axis=1, keepdims=True)
        return jnp.where(cnt >= k, cand, T)

    T = lax.fori_loop(0, 31, step, jnp.zeros((B, 1), jnp.int32))
    tval = lax.bitcast_convert_type(T, jnp.float32)

    gt = u > T
    c_gt = jnp.sum(gt.astype(jnp.int32), axis=1, keepdims=True)
    sum_gt = jnp.sum(jnp.where(gt, cls_loss, 0.0), axis=1, keepdims=True)
    neg_sum = jnp.sum(sum_gt + (k - c_gt).astype(jnp.float32) * tval)

    total_pos = jnp.maximum(jnp.sum(num_pos).astype(jnp.float32), 1.0)
    res = (pos_sum + neg_sum) / total_pos
    out_ref[...] = jnp.reshape(res, (1, 1))


def kernel(cls_preds, cls_targets):
    tgt = cls_targets.astype(jnp.int32)
    pt = jnp.transpose(cls_preds, (2, 0, 1))
    out = pl.pallas_call(
        _ohem_body,
        out_shape=jax.ShapeDtypeStruct((1, 1), jnp.float32),
    )(pt, tgt)
    return out[0, 0]


# speedup vs baseline: 64.5858x; 1.5311x over previous
import jax
import jax.numpy as jnp
from jax import lax
from jax.experimental import pallas as pl

NEG2POS_RATIO = 3


def _ohem_body(pt_ref, tgt_ref, out_ref):
    B, N = tgt_ref.shape
    p0 = pt_ref[0]
    p1 = pt_ref[1]
    t = tgt_ref[...]
    pos = t == 1

    d = p1 - p0
    s = jnp.where(pos, d, -d)
    ce = jnp.maximum(-s, 0.0) + jnp.log1p(jnp.exp(-jnp.abs(s)))

    num_pos = jnp.sum(pos.astype(jnp.int32), axis=1, keepdims=True)
    pos_sum = jnp.sum(jnp.where(pos, ce, 0.0))
    cls_loss = jnp.where(pos, 0.0, ce)
    u = lax.bitcast_convert_type(cls_loss, jnp.int32)
    k = jnp.clip(NEG2POS_RATIO * num_pos, 1, N - 1)

    cpos = jnp.sum((u > 0).astype(jnp.int32), axis=1, keepdims=True)
    shortcut = jnp.all(k >= cpos)

    def fast(_):
        return jnp.sum(cls_loss)

    def slow(_):
        def step(i, T):
            bit = 30 - i
            cand = T | lax.shift_left(jnp.int32(1), bit)
            cnt = jnp.sum((u >= cand).astype(jnp.int32), axis=1, keepdims=True)
            return jnp.where(cnt >= k, cand, T)

        T = lax.fori_loop(0, 31, step, jnp.zeros((B, 1), jnp.int32))
        tval = lax.bitcast_convert_type(T, jnp.float32)
        gt = u > T
        c_gt = jnp.sum(gt.astype(jnp.int32), axis=1, keepdims=True)
        sum_gt = jnp.sum(jnp.where(gt, cls_loss, 0.0), axis=1, keepdims=True)
        return jnp.sum(sum_gt + (k - c_gt).astype(jnp.float32) * tval)

    neg_sum = lax.cond(shortcut, fast, slow, None)

    total_pos = jnp.maximum(jnp.sum(num_pos).astype(jnp.float32), 1.0)
    res = (pos_sum + neg_sum) / total_pos
    out_ref[...] = jnp.reshape(res, (1, 1))


def kernel(cls_preds, cls_targets):
    tgt = cls_targets.astype(jnp.int32)
    pt = jnp.transpose(cls_preds, (2, 0, 1))
    out = pl.pallas_call(
        _ohem_body,
        out_shape=jax.ShapeDtypeStruct((1, 1), jnp.float32),
    )(pt, tgt)
    return out[0, 0]
